# Initial kernel scaffold; baseline (speedup 1.0000x reference)
import jax
import jax.numpy as jnp
from jax.experimental import pallas as pl


def kernel(node_feat, edge_feat, edge_index, W1, b1, W2, b2):
    raise NotImplementedError("write your pallas kernel here")


# speedup vs baseline: 3.5376x; 3.5376x over previous
import functools

import jax
import jax.numpy as jnp
from jax import lax
from jax.experimental import pallas as pl
from jax.experimental.pallas import tpu as pltpu
from jax.experimental.pallas import tpu_sc as plsc

_N = 10000
_E = 320000
_D = 128

_NC = 2
_NS = 16
_NW = _NC * _NS
_C = 80
_EPW = _E // _NW
_NCHUNK = _EPW // _C
_NP = 10240
_RPT = _NP // _NS

_mesh = plsc.VectorSubcoreMesh(core_axis_name="c", subcore_axis_name="s")


@functools.partial(
    pl.kernel,
    mesh=_mesh,
    out_type=jax.ShapeDtypeStruct((_NC * _NP, _D), jnp.float32),
    scratch_types=[
        pltpu.VMEM((_C,), jnp.int32),
        pltpu.VMEM((_C,), jnp.int32),
        pltpu.VMEM((_C, _D), jnp.float32),
        pltpu.VMEM((_C, _D), jnp.float32),
        pltpu.VMEM_SHARED((_NP, _D), jnp.float32),
        pltpu.SemaphoreType.DMA,
    ],
)
def _sc_scatter(src_hbm, dst_hbm, node_hbm, edge_hbm, zero_hbm, out_hbm,
                src_v, dst_v, rows_v, edge_v, acc, sem):
    cid = lax.axis_index("c")
    sid = lax.axis_index("s")
    wid = cid * _NS + sid

    r0 = pl.multiple_of(sid * _RPT, 8)
    pltpu.sync_copy(zero_hbm.at[pl.ds(0, _RPT)], acc.at[pl.ds(r0, _RPT)])
    plsc.subcore_barrier()

    base0 = wid * _EPW

    def body(j, carry):
        base = base0 + j * _C
        pltpu.sync_copy(src_hbm.at[pl.ds(base, _C)], src_v)
        pltpu.sync_copy(dst_hbm.at[pl.ds(base, _C)], dst_v)
        pltpu.async_copy(node_hbm.at[src_v], rows_v, sem).wait()
        pltpu.sync_copy(edge_hbm.at[pl.ds(base, _C)], edge_v)
        pltpu.sync_copy(rows_v, acc.at[dst_v], add=True)
        pltpu.sync_copy(edge_v, acc.at[dst_v], add=True)
        return carry

    lax.fori_loop(0, _NCHUNK, body, 0)
    plsc.subcore_barrier()

    out_r0 = pl.multiple_of(cid * _NP + r0, 8)
    pltpu.sync_copy(acc.at[pl.ds(r0, _RPT)], out_hbm.at[pl.ds(out_r0, _RPT)])


def _mlp_block(p0_ref, p1_ref, w1_ref, b1_ref, w2_ref, b2_ref, o_ref):
    x = p0_ref[...] + p1_ref[...]
    h = jnp.dot(x, w1_ref[...], preferred_element_type=jnp.float32)
    h = jnp.maximum(h + b1_ref[...], 0.0)
    o = jnp.dot(h, w2_ref[...], preferred_element_type=jnp.float32)
    o_ref[...] = o + b2_ref[...]


def kernel(node_feat, edge_feat, edge_index, W1, b1, W2, b2):
    src = edge_index[0]
    dst = edge_index[1]
    zeros = jnp.zeros((_RPT, _D), jnp.float32)
    part = _sc_scatter(src, dst, node_feat, edge_feat, zeros)

    R = 512
    nblk = _NP // R
    out = pl.pallas_call(
        _mlp_block,
        grid=(nblk,),
        in_specs=[
            pl.BlockSpec((R, _D), lambda i: (i, 0)),
            pl.BlockSpec((R, _D), lambda i: (i + nblk, 0)),
            pl.BlockSpec((_D, 2 * _D), lambda i: (0, 0)),
            pl.BlockSpec((1, 2 * _D), lambda i: (0, 0)),
            pl.BlockSpec((2 * _D, _D), lambda i: (0, 0)),
            pl.BlockSpec((1, _D), lambda i: (0, 0)),
        ],
        out_specs=pl.BlockSpec((R, _D), lambda i: (i, 0)),
        out_shape=jax.ShapeDtypeStruct((_NP, _D), jnp.float32),
    )(part, part, W1, b1.reshape(1, -1), W2, b2.reshape(1, -1))
    return out[:_N]


# speedup vs baseline: 7.1051x; 2.0084x over previous
import functools

import jax
import jax.numpy as jnp
from jax import lax
from jax.experimental import pallas as pl
from jax.experimental.pallas import tpu as pltpu
from jax.experimental.pallas import tpu_sc as plsc

_N = 10000
_E = 320000
_D = 128

_NC = 2
_NS = 16
_NW = _NC * _NS
_C = 80
_EPW = _E // _NW
_NCHUNK = _EPW // _C
_NP = 10240
_RPT = _NP // _NS

_mesh = plsc.VectorSubcoreMesh(core_axis_name="c", subcore_axis_name="s")


@functools.partial(
    pl.kernel,
    mesh=_mesh,
    out_type=jax.ShapeDtypeStruct((_NC * _NP, _D), jnp.float32),
    scratch_types=[
        pltpu.VMEM((_C,), jnp.int32),
        pltpu.VMEM((_C,), jnp.int32),
        pltpu.VMEM((_C,), jnp.int32),
        pltpu.VMEM((_C,), jnp.int32),
        pltpu.VMEM((_C, _D), jnp.float32),
        pltpu.VMEM((_C, _D), jnp.float32),
        pltpu.VMEM((_C, _D), jnp.float32),
        pltpu.VMEM((_C, _D), jnp.float32),
        pltpu.VMEM_SHARED((_NP, _D), jnp.float32),
        pltpu.SemaphoreType.DMA,
        pltpu.SemaphoreType.DMA,
        pltpu.SemaphoreType.DMA,
        pltpu.SemaphoreType.DMA,
        pltpu.SemaphoreType.DMA,
        pltpu.SemaphoreType.DMA,
    ],
)
def _sc_scatter(src_hbm, dst_hbm, node_hbm, edge_hbm, zero_hbm, out_hbm,
                src_v0, dst_v0, src_v1, dst_v1,
                rows_v0, edge_v0, rows_v1, edge_v1,
                acc, sem_i0, sem_i1, sem_ge0, sem_ge1, sem_s0, sem_s1):
    cid = lax.axis_index("c")
    sid = lax.axis_index("s")
    wid = cid * _NS + sid

    r0 = pl.multiple_of(sid * _RPT, 8)
    pltpu.sync_copy(zero_hbm.at[pl.ds(0, _RPT)], acc.at[pl.ds(r0, _RPT)])
    plsc.subcore_barrier()

    base0 = wid * _EPW
    bufs = ((src_v0, dst_v0, rows_v0, edge_v0, sem_i0, sem_ge0, sem_s0),
            (src_v1, dst_v1, rows_v1, edge_v1, sem_i1, sem_ge1, sem_s1))

    def issue_idx(j, b):
        base = pl.multiple_of(base0 + j * _C, 8)
        sv, dv, _, _, si, _, _ = bufs[b]
        pltpu.async_copy(src_hbm.at[pl.ds(base, _C)], sv, si)
        pltpu.async_copy(dst_hbm.at[pl.ds(base, _C)], dv, si)

    def wait_idx(j, b):
        base = pl.multiple_of(base0 + j * _C, 8)
        sv, dv, _, _, si, _, _ = bufs[b]
        pltpu.make_async_copy(src_hbm.at[pl.ds(base, _C)], sv, si).wait()
        pltpu.make_async_copy(dst_hbm.at[pl.ds(base, _C)], dv, si).wait()

    def wait_scatter(b):
        _, dv, rv, ev, _, _, ss = bufs[b]
        pltpu.make_async_copy(rv, acc.at[dv], ss).wait()
        pltpu.make_async_copy(ev, acc.at[dv], ss).wait()

    def step(j, b, wait_prev, next_j):
        sv, dv, rv, ev, si, sge, ss = bufs[b]
        base = pl.multiple_of(base0 + j * _C, 8)
        wait_idx(j, b)
        cg = pltpu.async_copy(node_hbm.at[sv], rv, sge)
        ce = pltpu.async_copy(edge_hbm.at[pl.ds(base, _C)], ev, sge)
        if wait_prev:
            wait_scatter(1 - b)
        if next_j is not None:
            issue_idx(next_j, 1 - b)
        cg.wait()
        ce.wait()
        pltpu.async_copy(rv, acc.at[dv], ss, add=True)
        pltpu.async_copy(ev, acc.at[dv], ss, add=True)

    issue_idx(0, 0)
    step(0, 0, wait_prev=False, next_j=1)
    step(1, 1, wait_prev=True, next_j=2)

    def pair_body(t, carry):
        j0 = 2 * t
        step(j0, 0, wait_prev=True, next_j=j0 + 1)
        step(j0 + 1, 1, wait_prev=True, next_j=j0 + 2)
        return carry

    lax.fori_loop(1, _NCHUNK // 2, pair_body, 0)
    step(_NCHUNK - 1, 0, wait_prev=True, next_j=None)
    wait_scatter(0)
    plsc.subcore_barrier()

    out_r0 = pl.multiple_of(cid * _NP + r0, 8)
    pltpu.sync_copy(acc.at[pl.ds(r0, _RPT)], out_hbm.at[pl.ds(out_r0, _RPT)])


def _mlp_block(p0_ref, p1_ref, w1_ref, b1_ref, w2_ref, b2_ref, o_ref):
    x = p0_ref[...] + p1_ref[...]
    h = jnp.dot(x, w1_ref[...], preferred_element_type=jnp.float32)
    h = jnp.maximum(h + b1_ref[...], 0.0)
    o = jnp.dot(h, w2_ref[...], preferred_element_type=jnp.float32)
    o_ref[...] = o + b2_ref[...]


def kernel(node_feat, edge_feat, edge_index, W1, b1, W2, b2):
    src = edge_index[0]
    dst = edge_index[1]
    zeros = jnp.zeros((_RPT, _D), jnp.float32)
    part = _sc_scatter(src, dst, node_feat, edge_feat, zeros)

    R = 512
    nblk = _NP // R
    out = pl.pallas_call(
        _mlp_block,
        grid=(nblk,),
        in_specs=[
            pl.BlockSpec((R, _D), lambda i: (i, 0)),
            pl.BlockSpec((R, _D), lambda i: (i + nblk, 0)),
            pl.BlockSpec((_D, 2 * _D), lambda i: (0, 0)),
            pl.BlockSpec((1, 2 * _D), lambda i: (0, 0)),
            pl.BlockSpec((2 * _D, _D), lambda i: (0, 0)),
            pl.BlockSpec((1, _D), lambda i: (0, 0)),
        ],
        out_specs=pl.BlockSpec((R, _D), lambda i: (i, 0)),
        out_shape=jax.ShapeDtypeStruct((_NP, _D), jnp.float32),
    )(part, part, W1, b1.reshape(1, -1), W2, b2.reshape(1, -1))
    return out[:_N]


# speedup vs baseline: 8.4430x; 1.1883x over previous
import functools

import jax
import jax.numpy as jnp
from jax import lax
from jax.experimental import pallas as pl
from jax.experimental.pallas import tpu as pltpu
from jax.experimental.pallas import tpu_sc as plsc

_N = 10000
_E = 320000
_D = 128

_NC = 2
_NS = 16
_NW = _NC * _NS
_C = 40
_EPW = _E // _NW
_NCHUNK = _EPW // _C
_RPT = 632
_RPT_LAST = _N - 15 * _RPT

_mesh = plsc.VectorSubcoreMesh(core_axis_name="c", subcore_axis_name="s")


@functools.partial(
    pl.kernel,
    mesh=_mesh,
    out_type=jax.ShapeDtypeStruct((_NC * _N, _D), jnp.float32),
    scratch_types=(
        [pltpu.VMEM((_EPW,), jnp.int32) for _ in range(2)]
        + [pltpu.VMEM((_C, _D), jnp.float32) for _ in range(6)]
        + [pltpu.VMEM_SHARED((_N, _D), jnp.float32)]
        + [pltpu.SemaphoreType.DMA for _ in range(7)]
    ),
)
def _sc_scatter(src_hbm, dst_hbm, node_hbm, edge_hbm, zero_hbm, out_hbm,
                srcst, dstst,
                rv0, ev0, rv1, ev1, rv2, ev2,
                acc,
                sst, sg0, sg1, sg2, ss0, ss1, ss2):
    cid = lax.axis_index("c")
    sid = lax.axis_index("s")
    wid = cid * _NS + sid
    base0 = pl.multiple_of(wid * _EPW, 8)

    pltpu.async_copy(src_hbm.at[pl.ds(base0, _EPW)], srcst, sst)
    pltpu.async_copy(dst_hbm.at[pl.ds(base0, _EPW)], dstst, sst)
    r0 = pl.multiple_of(sid * _RPT, 8)

    @pl.when(sid < _NS - 1)
    def _():
        pltpu.sync_copy(zero_hbm.at[pl.ds(0, _RPT)], acc.at[pl.ds(r0, _RPT)])

    @pl.when(sid == _NS - 1)
    def _():
        pltpu.sync_copy(zero_hbm.at[pl.ds(0, _RPT_LAST)],
                        acc.at[pl.ds(15 * _RPT, _RPT_LAST)])

    pltpu.make_async_copy(src_hbm.at[pl.ds(base0, _EPW)], srcst, sst).wait()
    pltpu.make_async_copy(dst_hbm.at[pl.ds(base0, _EPW)], dstst, sst).wait()
    plsc.subcore_barrier()

    bufs = ((rv0, ev0, sg0, ss0), (rv1, ev1, sg1, ss1), (rv2, ev2, sg2, ss2))

    def G(j, p):
        base = pl.multiple_of(base0 + j * _C, 8)
        rv, ev, sg, _ = bufs[p]
        pltpu.async_copy(node_hbm.at[srcst.at[pl.ds(j * _C, _C)]], rv, sg)
        pltpu.async_copy(edge_hbm.at[pl.ds(base, _C)], ev, sg)

    def WG(j, p):
        base = pl.multiple_of(base0 + j * _C, 8)
        rv, ev, sg, _ = bufs[p]
        pltpu.make_async_copy(node_hbm.at[srcst.at[pl.ds(j * _C, _C)]], rv, sg).wait()
        pltpu.make_async_copy(edge_hbm.at[pl.ds(base, _C)], ev, sg).wait()

    def S(j, p):
        rv, ev, _, ss = bufs[p]
        pltpu.async_copy(rv, acc.at[dstst.at[pl.ds(j * _C, _C)]], ss, add=True)
        pltpu.async_copy(ev, acc.at[dstst.at[pl.ds(j * _C, _C)]], ss, add=True)

    def WS(j, p):
        rv, ev, _, ss = bufs[p]
        pltpu.make_async_copy(rv, acc.at[dstst.at[pl.ds(j * _C, _C)]], ss).wait()
        pltpu.make_async_copy(ev, acc.at[dstst.at[pl.ds(j * _C, _C)]], ss).wait()

    def iter_j(j, p, wait_sc=True):
        if wait_sc:
            WS(j - 3, p)
        G(j, p)
        WG(j - 1, (p + 2) % 3)
        S(j - 1, (p + 2) % 3)

    G(0, 0)
    iter_j(1, 1, wait_sc=False)
    iter_j(2, 2, wait_sc=False)

    def tri_body(t, carry):
        j0 = 3 * t
        iter_j(j0, 0)
        iter_j(j0 + 1, 1)
        iter_j(j0 + 2, 2)
        return carry

    lax.fori_loop(1, _NCHUNK // 3, tri_body, 0)
    for j in range(3 * (_NCHUNK // 3), _NCHUNK):
        iter_j(j, j % 3)
    jl = _NCHUNK - 1
    WG(jl, jl % 3)
    S(jl, jl % 3)
    WS(jl - 2, (jl - 2) % 3)
    WS(jl - 1, (jl - 1) % 3)
    WS(jl, jl % 3)
    plsc.subcore_barrier()

    @pl.when(sid < _NS - 1)
    def _():
        out_r0 = pl.multiple_of(cid * _N + r0, 8)
        pltpu.sync_copy(acc.at[pl.ds(r0, _RPT)], out_hbm.at[pl.ds(out_r0, _RPT)])

    @pl.when(sid == _NS - 1)
    def _():
        out_l0 = pl.multiple_of(cid * _N + 15 * _RPT, 8)
        pltpu.sync_copy(acc.at[pl.ds(15 * _RPT, _RPT_LAST)],
                        out_hbm.at[pl.ds(out_l0, _RPT_LAST)])


def _mlp_block(p0_ref, p1_ref, w1_ref, b1_ref, w2_ref, b2_ref, o_ref):
    x = p0_ref[...] + p1_ref[...]
    h = jnp.dot(x, w1_ref[...], preferred_element_type=jnp.float32)
    h = jnp.maximum(h + b1_ref[...], 0.0)
    o = jnp.dot(h, w2_ref[...], preferred_element_type=jnp.float32)
    o_ref[...] = o + b2_ref[...]


def kernel(node_feat, edge_feat, edge_index, W1, b1, W2, b2):
    src = edge_index[0]
    dst = edge_index[1]
    zeros = jnp.zeros((_RPT, _D), jnp.float32)
    part = _sc_scatter(src, dst, node_feat, edge_feat, zeros)

    R = 400
    nblk = _N // R
    out = pl.pallas_call(
        _mlp_block,
        grid=(nblk,),
        in_specs=[
            pl.BlockSpec((R, _D), lambda i: (i, 0)),
            pl.BlockSpec((R, _D), lambda i: (i + nblk, 0)),
            pl.BlockSpec((_D, 2 * _D), lambda i: (0, 0)),
            pl.BlockSpec((1, 2 * _D), lambda i: (0, 0)),
            pl.BlockSpec((2 * _D, _D), lambda i: (0, 0)),
            pl.BlockSpec((1, _D), lambda i: (0, 0)),
        ],
        out_specs=pl.BlockSpec((R, _D), lambda i: (i, 0)),
        out_shape=jax.ShapeDtypeStruct((_N, _D), jnp.float32),
    )(part, part, W1, b1.reshape(1, -1), W2, b2.reshape(1, -1))
    return out
